# baseline (speedup 1.0000x reference)
import functools

import jax
import jax.numpy as jnp
import numpy as np
from jax import lax
from jax.experimental import pallas as pl
from jax.experimental.pallas import tpu as pltpu
from jax.experimental.pallas import tpu_sc as plsc

_B, _H, _W, _C = 2, 192, 192, 96
_P = _H * _W
_R = (_H * 2) * (_W * 2)
_HALF = _R // 2
_CHUNK = 4608
_NPAIRS = _B * _C
_NWORK = 32
_TASKS_PER_W = _NPAIRS // _NWORK

_THIRD = np.float32(1.0 / 3.0)


def _sc_unpool(mask_t, upd_t):
    mesh = plsc.VectorSubcoreMesh(core_axis_name="c", subcore_axis_name="s")

    @functools.partial(
        pl.kernel,
        mesh=mesh,
        out_type=jax.ShapeDtypeStruct((_NPAIRS, _R), jnp.float32),
        scratch_types=[
            pltpu.VMEM((_HALF,), jnp.float32),
            pltpu.VMEM((_P,), jnp.int32),
            pltpu.VMEM((2, _CHUNK), jnp.int32),
            pltpu.VMEM((2, _CHUNK), jnp.float32),
            pltpu.VMEM_SHARED((7 * _CHUNK,), jnp.float32),
            pltpu.SemaphoreType.DMA,
            pltpu.SemaphoreType.DMA,
            pltpu.SemaphoreType.DMA,
            pltpu.SemaphoreType.DMA,
        ],
        compiler_params=pltpu.CompilerParams(needs_layout_passes=False),
    )
    def k(mask_hbm, upd_hbm, out_hbm, acc, rcache, mbuf, ubuf, ucache,
          sem_a, sem_b, out_sem, spm_sem):
        sid = lax.axis_index("s")
        wid = sid * 2 + lax.axis_index("c")
        sems = (sem_a, sem_b)
        nchunks = _P // _CHUNK
        zeros = jnp.zeros((16,), jnp.float32)

        def task(j, carry):
            pair = wid * _TASKS_PER_W + j

            def issue(ck, half):
                par = ck % 2
                off = ck * _CHUNK
                if half == 0:
                    cu = pltpu.async_copy(
                        upd_hbm.at[pair, pl.ds(off, _CHUNK)], ubuf.at[par],
                        sems[par])
                    cm = pltpu.async_copy(
                        mask_hbm.at[pair, pl.ds(off, _CHUNK)], mbuf.at[par],
                        sems[par])
                    return cm, cu
                if ck < 7:
                    cu = pltpu.async_copy(
                        ucache.at[pl.ds(off, _CHUNK)], ubuf.at[par],
                        sems[par])
                else:
                    cu = pltpu.async_copy(
                        upd_hbm.at[pair, pl.ds(off, _CHUNK)], ubuf.at[par],
                        sems[par])
                return (cu,)

            for half in (0, 1):
                lo = half * _HALF
                cps = {0: issue(0, half)}

                drain = pltpu.make_async_copy(
                    acc, out_hbm.at[pair, pl.ds(lo, _HALF)], out_sem)
                if half == 0:
                    @pl.when(j > 0)
                    def _():
                        drain.wait()
                else:
                    drain.wait()

                @plsc.parallel_loop(0, _HALF // 16, unroll=8)
                def _(i):
                    acc[pl.ds(i * 16, 16)] = zeros

                for ck in range(nchunks):
                    par = ck % 2
                    coff = ck * _CHUNK
                    if ck + 1 < nchunks:
                        cps[ck + 1] = issue(ck + 1, half)
                    for cp in cps.pop(ck):
                        cp.wait()

                    if half == 0:
                        if ck < 7:
                            spm_cp = pltpu.async_copy(
                                ubuf.at[par], ucache.at[pl.ds(coff, _CHUNK)],
                                spm_sem)

                        @plsc.parallel_loop(0, _CHUNK // 16, unroll=8)
                        def _(g, par=par, coff=coff):
                            s = g * 16
                            m = mbuf[par, pl.ds(s, 16)]
                            v = ubuf[par, pl.ds(s, 16)]
                            t5 = lax.shift_right_logical(m, 5)
                            r = (t5.astype(jnp.float32) * _THIRD).astype(
                                jnp.int32)
                            rcache[pl.ds(coff + s, 16)] = r
                            ok = r < _HALF
                            plsc.addupdate_scatter(acc, [r], v, mask=ok)

                        if ck < 7:
                            spm_cp.wait()
                    else:
                        @plsc.parallel_loop(0, _CHUNK // 16, unroll=8)
                        def _(g, par=par, coff=coff):
                            s = g * 16
                            r = rcache[pl.ds(coff + s, 16)]
                            v = ubuf[par, pl.ds(s, 16)]
                            idx = r - _HALF
                            ok = r >= _HALF
                            plsc.addupdate_scatter(acc, [idx], v, mask=ok)

                pltpu.async_copy(
                    acc, out_hbm.at[pair, pl.ds(lo, _HALF)], out_sem)

            return carry

        lax.fori_loop(0, _TASKS_PER_W, task, 0)
        pltpu.make_async_copy(
            acc, out_hbm.at[0, pl.ds(0, _HALF)], out_sem).wait()

    return k(mask_t, upd_t)


def kernel(updates, mask):
    B, H, W, C = updates.shape
    Ho, Wo = H * 2, W * 2
    m32 = mask.astype(jnp.int32)
    u_t = updates.reshape(B, H * W, C).transpose(0, 2, 1).reshape(B * C, H * W)
    m_t = m32.reshape(B, H * W, C).transpose(0, 2, 1).reshape(B * C, H * W)
    out = _sc_unpool(m_t, u_t)
    return out.reshape(B, C, Ho * Wo).transpose(0, 2, 1).reshape(B, Ho, Wo, C)


# speedup vs baseline: 1.0020x; 1.0020x over previous
import functools

import jax
import jax.numpy as jnp
import numpy as np
from jax import lax
from jax.experimental import pallas as pl
from jax.experimental.pallas import tpu as pltpu
from jax.experimental.pallas import tpu_sc as plsc

_B, _H, _W, _C = 2, 192, 192, 96
_P = _H * _W
_R = (_H * 2) * (_W * 2)
_HALF = _R // 2
_CHUNK = 4608
_NPAIRS = _B * _C
_NWORK = 32
_TASKS_PER_W = _NPAIRS // _NWORK

_THIRD = np.float32(1.0 / 3.0)


def _sc_unpool(mask_t, upd_t):
    mesh = plsc.VectorSubcoreMesh(core_axis_name="c", subcore_axis_name="s")

    @functools.partial(
        pl.kernel,
        mesh=mesh,
        out_type=jax.ShapeDtypeStruct((_NPAIRS, _R), jnp.float32),
        scratch_types=[
            pltpu.VMEM((_HALF,), jnp.float32),
            pltpu.VMEM((_P,), jnp.int32),
            pltpu.VMEM((2, _CHUNK), jnp.int32),
            pltpu.VMEM((2, _CHUNK), jnp.float32),
            pltpu.SemaphoreType.DMA,
            pltpu.SemaphoreType.DMA,
            pltpu.SemaphoreType.DMA,
        ],
        compiler_params=pltpu.CompilerParams(needs_layout_passes=False),
    )
    def k(mask_hbm, upd_hbm, out_hbm, acc, rcache, mbuf, ubuf,
          sem_a, sem_b, out_sem):
        wid = lax.axis_index("s") * 2 + lax.axis_index("c")
        sems = (sem_a, sem_b)
        nchunks = _P // _CHUNK
        zeros = jnp.zeros((16,), jnp.float32)

        def task(j, carry):
            pair = wid * _TASKS_PER_W + j

            def issue(ck, half):
                par = ck % 2
                off = ck * _CHUNK
                if half == 0:
                    cu = pltpu.async_copy(
                        upd_hbm.at[pair, pl.ds(off, _CHUNK)], ubuf.at[par],
                        sems[par])
                    cm = pltpu.async_copy(
                        mask_hbm.at[pair, pl.ds(off, _CHUNK)], mbuf.at[par],
                        sems[par])
                    return cm, cu
                cu = pltpu.async_copy(
                    upd_hbm.at[pair, pl.ds(off, _CHUNK)], ubuf.at[par],
                    sems[par])
                return (cu,)

            for half in (0, 1):
                lo = half * _HALF
                cps = {0: issue(0, half)}

                drain = pltpu.make_async_copy(
                    acc, out_hbm.at[pair, pl.ds(lo, _HALF)], out_sem)
                if half == 0:
                    @pl.when(j > 0)
                    def _():
                        drain.wait()
                else:
                    drain.wait()

                @plsc.parallel_loop(0, _HALF // 16, unroll=8)
                def _(i):
                    acc[pl.ds(i * 16, 16)] = zeros

                for ck in range(nchunks):
                    par = ck % 2
                    coff = ck * _CHUNK
                    if ck + 1 < nchunks:
                        cps[ck + 1] = issue(ck + 1, half)
                    for cp in cps.pop(ck):
                        cp.wait()

                    if half == 0:
                        @plsc.parallel_loop(0, _CHUNK // 16, unroll=16)
                        def _(g, par=par, coff=coff):
                            s = g * 16
                            m = mbuf[par, pl.ds(s, 16)]
                            v = ubuf[par, pl.ds(s, 16)]
                            t5 = lax.shift_right_logical(m, 5)
                            r = (t5.astype(jnp.float32) * _THIRD).astype(
                                jnp.int32)
                            rcache[pl.ds(coff + s, 16)] = r
                            ok = r < _HALF
                            plsc.addupdate_scatter(acc, [r], v, mask=ok)

                    else:
                        @plsc.parallel_loop(0, _CHUNK // 16, unroll=16)
                        def _(g, par=par, coff=coff):
                            s = g * 16
                            r = rcache[pl.ds(coff + s, 16)]
                            v = ubuf[par, pl.ds(s, 16)]
                            idx = r - _HALF
                            ok = r >= _HALF
                            plsc.addupdate_scatter(acc, [idx], v, mask=ok)

                pltpu.async_copy(
                    acc, out_hbm.at[pair, pl.ds(lo, _HALF)], out_sem)

            return carry

        lax.fori_loop(0, _TASKS_PER_W, task, 0)
        pltpu.make_async_copy(
            acc, out_hbm.at[0, pl.ds(0, _HALF)], out_sem).wait()

    return k(mask_t, upd_t)


def kernel(updates, mask):
    B, H, W, C = updates.shape
    Ho, Wo = H * 2, W * 2
    m32 = mask.astype(jnp.int32)
    u_t = updates.reshape(B, H * W, C).transpose(0, 2, 1).reshape(B * C, H * W)
    m_t = m32.reshape(B, H * W, C).transpose(0, 2, 1).reshape(B * C, H * W)
    out = _sc_unpool(m_t, u_t)
    return out.reshape(B, C, Ho * Wo).transpose(0, 2, 1).reshape(B, Ho, Wo, C)


# speedup vs baseline: 1.0657x; 1.0636x over previous
import functools

import jax
import jax.numpy as jnp
import numpy as np
from jax import lax
from jax.experimental import pallas as pl
from jax.experimental.pallas import tpu as pltpu
from jax.experimental.pallas import tpu_sc as plsc

_B, _H, _W, _C = 2, 192, 192, 96
_P = _H * _W
_R = (_H * 2) * (_W * 2)
_HALF = _R // 2
_CHUNK = 3072
_NBUF = 3
_NPAIRS = _B * _C
_NWORK = 32
_TASKS_PER_W = _NPAIRS // _NWORK

_THIRD = np.float32(1.0 / 3.0)


def _sc_unpool(mask_t, upd_t):
    mesh = plsc.VectorSubcoreMesh(core_axis_name="c", subcore_axis_name="s")

    @functools.partial(
        pl.kernel,
        mesh=mesh,
        out_type=jax.ShapeDtypeStruct((_NPAIRS, _R), jnp.float32),
        scratch_types=[
            pltpu.VMEM((_HALF,), jnp.float32),
            pltpu.VMEM((_P,), jnp.int32),
            pltpu.VMEM((_CHUNK,), jnp.int32),
            pltpu.VMEM((_CHUNK,), jnp.int32),
            pltpu.VMEM((_CHUNK,), jnp.int32),
            pltpu.VMEM((_CHUNK,), jnp.float32),
            pltpu.VMEM((_CHUNK,), jnp.float32),
            pltpu.VMEM((_CHUNK,), jnp.float32),
            pltpu.SemaphoreType.DMA,
            pltpu.SemaphoreType.DMA,
            pltpu.SemaphoreType.DMA,
            pltpu.SemaphoreType.DMA,
        ],
        compiler_params=pltpu.CompilerParams(needs_layout_passes=False),
    )
    def k(mask_hbm, upd_hbm, out_hbm, acc, rcache, mb0, mb1, mb2,
          ub0, ub1, ub2, sem_a, sem_b, sem_c, out_sem):
        wid = lax.axis_index("s") * 2 + lax.axis_index("c")
        sems = (sem_a, sem_b, sem_c)
        mbufs = (mb0, mb1, mb2)
        ubufs = (ub0, ub1, ub2)
        nchunks = _P // _CHUNK
        zeros = jnp.zeros((16,), jnp.float32)

        def task(j, carry):
            pair = wid * _TASKS_PER_W + j

            def issue(ck, half):
                par = ck % _NBUF
                off = ck * _CHUNK
                if half == 0:
                    cu = pltpu.async_copy(
                        upd_hbm.at[pair, pl.ds(off, _CHUNK)], ubufs[par],
                        sems[par])
                    cm = pltpu.async_copy(
                        mask_hbm.at[pair, pl.ds(off, _CHUNK)], mbufs[par],
                        sems[par])
                    return cm, cu
                cu = pltpu.async_copy(
                    upd_hbm.at[pair, pl.ds(off, _CHUNK)], ubufs[par],
                    sems[par])
                return (cu,)

            for half in (0, 1):
                lo = half * _HALF
                cps = {0: issue(0, half), 1: issue(1, half)}

                drain = pltpu.make_async_copy(
                    acc, out_hbm.at[pair, pl.ds(lo, _HALF)], out_sem)
                if half == 0:
                    @pl.when(j > 0)
                    def _():
                        drain.wait()
                else:
                    drain.wait()

                @plsc.parallel_loop(0, _HALF // 16, unroll=8)
                def _(i):
                    acc[pl.ds(i * 16, 16)] = zeros

                for ck in range(nchunks):
                    par = ck % _NBUF
                    coff = ck * _CHUNK
                    if ck + 2 < nchunks:
                        cps[ck + 2] = issue(ck + 2, half)
                    for cp in cps.pop(ck):
                        cp.wait()

                    if half == 0:
                        @plsc.parallel_loop(0, _CHUNK // 16, unroll=8)
                        def _(g, par=par, coff=coff):
                            s = g * 16
                            m = mbufs[par][pl.ds(s, 16)]
                            v = ubufs[par][pl.ds(s, 16)]
                            t5 = lax.shift_right_logical(m, 5)
                            r = (t5.astype(jnp.float32) * _THIRD).astype(
                                jnp.int32)
                            rcache[pl.ds(coff + s, 16)] = r
                            ok = r < _HALF
                            plsc.addupdate_scatter(acc, [r], v, mask=ok)

                    else:
                        @plsc.parallel_loop(0, _CHUNK // 16, unroll=8)
                        def _(g, par=par, coff=coff):
                            s = g * 16
                            r = rcache[pl.ds(coff + s, 16)]
                            v = ubufs[par][pl.ds(s, 16)]
                            idx = r - _HALF
                            ok = r >= _HALF
                            plsc.addupdate_scatter(acc, [idx], v, mask=ok)

                pltpu.async_copy(
                    acc, out_hbm.at[pair, pl.ds(lo, _HALF)], out_sem)

            return carry

        lax.fori_loop(0, _TASKS_PER_W, task, 0)
        pltpu.make_async_copy(
            acc, out_hbm.at[0, pl.ds(0, _HALF)], out_sem).wait()

    return k(mask_t, upd_t)


def kernel(updates, mask):
    B, H, W, C = updates.shape
    Ho, Wo = H * 2, W * 2
    m32 = mask.astype(jnp.int32)
    u_t = updates.reshape(B, H * W, C).transpose(0, 2, 1).reshape(B * C, H * W)
    m_t = m32.reshape(B, H * W, C).transpose(0, 2, 1).reshape(B * C, H * W)
    out = _sc_unpool(m_t, u_t)
    return out.reshape(B, C, Ho * Wo).transpose(0, 2, 1).reshape(B, Ho, Wo, C)


# speedup vs baseline: 1.0768x; 1.0104x over previous
import functools

import jax
import jax.numpy as jnp
import numpy as np
from jax import lax
from jax.experimental import pallas as pl
from jax.experimental.pallas import tpu as pltpu
from jax.experimental.pallas import tpu_sc as plsc

_B, _H, _W, _C = 2, 192, 192, 96
_P = _H * _W
_R = (_H * 2) * (_W * 2)
_HALF = _R // 2
_CHUNK = 2304
_NBUF = 4
_NPAIRS = _B * _C
_NWORK = 32
_TASKS_PER_W = _NPAIRS // _NWORK

_THIRD = np.float32(1.0 / 3.0)


def _sc_unpool(mask_t, upd_t):
    mesh = plsc.VectorSubcoreMesh(core_axis_name="c", subcore_axis_name="s")

    @functools.partial(
        pl.kernel,
        mesh=mesh,
        out_type=jax.ShapeDtypeStruct((_NPAIRS, _R), jnp.float32),
        scratch_types=[
            pltpu.VMEM((_HALF,), jnp.float32),
            pltpu.VMEM((_P,), jnp.int32),
            pltpu.VMEM((_CHUNK,), jnp.int32),
            pltpu.VMEM((_CHUNK,), jnp.int32),
            pltpu.VMEM((_CHUNK,), jnp.int32),
            pltpu.VMEM((_CHUNK,), jnp.int32),
            pltpu.VMEM((_CHUNK,), jnp.float32),
            pltpu.VMEM((_CHUNK,), jnp.float32),
            pltpu.VMEM((_CHUNK,), jnp.float32),
            pltpu.VMEM((_CHUNK,), jnp.float32),
            pltpu.SemaphoreType.DMA,
            pltpu.SemaphoreType.DMA,
            pltpu.SemaphoreType.DMA,
            pltpu.SemaphoreType.DMA,
            pltpu.SemaphoreType.DMA,
        ],
        compiler_params=pltpu.CompilerParams(needs_layout_passes=False),
    )
    def k(mask_hbm, upd_hbm, out_hbm, acc, rcache, mb0, mb1, mb2, mb3,
          ub0, ub1, ub2, ub3, sem_a, sem_b, sem_c, sem_d, out_sem):
        wid = lax.axis_index("s") * 2 + lax.axis_index("c")
        sems = (sem_a, sem_b, sem_c, sem_d)
        mbufs = (mb0, mb1, mb2, mb3)
        ubufs = (ub0, ub1, ub2, ub3)
        nchunks = _P // _CHUNK
        zeros = jnp.zeros((16,), jnp.float32)

        def task(j, carry):
            pair = wid * _TASKS_PER_W + j

            def issue(ck, half):
                par = ck % _NBUF
                off = ck * _CHUNK
                if half == 0:
                    cu = pltpu.async_copy(
                        upd_hbm.at[pair, pl.ds(off, _CHUNK)], ubufs[par],
                        sems[par])
                    cm = pltpu.async_copy(
                        mask_hbm.at[pair, pl.ds(off, _CHUNK)], mbufs[par],
                        sems[par])
                    return cm, cu
                cu = pltpu.async_copy(
                    upd_hbm.at[pair, pl.ds(off, _CHUNK)], ubufs[par],
                    sems[par])
                return (cu,)

            for half in (0, 1):
                lo = half * _HALF
                cps = {0: issue(0, half), 1: issue(1, half), 2: issue(2, half)}

                drain = pltpu.make_async_copy(
                    acc, out_hbm.at[pair, pl.ds(lo, _HALF)], out_sem)
                if half == 0:
                    @pl.when(j > 0)
                    def _():
                        drain.wait()
                else:
                    drain.wait()

                @plsc.parallel_loop(0, _HALF // 16, unroll=8)
                def _(i):
                    acc[pl.ds(i * 16, 16)] = zeros

                for ck in range(nchunks):
                    par = ck % _NBUF
                    coff = ck * _CHUNK
                    if ck + 3 < nchunks:
                        cps[ck + 3] = issue(ck + 3, half)
                    for cp in cps.pop(ck):
                        cp.wait()

                    if half == 0:
                        @plsc.parallel_loop(0, _CHUNK // 16, unroll=8)
                        def _(g, par=par, coff=coff):
                            s = g * 16
                            m = mbufs[par][pl.ds(s, 16)]
                            v = ubufs[par][pl.ds(s, 16)]
                            t5 = lax.shift_right_logical(m, 5)
                            r = (t5.astype(jnp.float32) * _THIRD).astype(
                                jnp.int32)
                            rcache[pl.ds(coff + s, 16)] = r
                            ok = r < _HALF
                            plsc.addupdate_scatter(acc, [r], v, mask=ok)

                    else:
                        @plsc.parallel_loop(0, _CHUNK // 16, unroll=8)
                        def _(g, par=par, coff=coff):
                            s = g * 16
                            r = rcache[pl.ds(coff + s, 16)]
                            v = ubufs[par][pl.ds(s, 16)]
                            idx = r - _HALF
                            ok = r >= _HALF
                            plsc.addupdate_scatter(acc, [idx], v, mask=ok)

                pltpu.async_copy(
                    acc, out_hbm.at[pair, pl.ds(lo, _HALF)], out_sem)

            return carry

        lax.fori_loop(0, _TASKS_PER_W, task, 0)
        pltpu.make_async_copy(
            acc, out_hbm.at[0, pl.ds(0, _HALF)], out_sem).wait()

    return k(mask_t, upd_t)


def kernel(updates, mask):
    B, H, W, C = updates.shape
    Ho, Wo = H * 2, W * 2
    m32 = mask.astype(jnp.int32)
    u_t = updates.reshape(B, H * W, C).transpose(0, 2, 1).reshape(B * C, H * W)
    m_t = m32.reshape(B, H * W, C).transpose(0, 2, 1).reshape(B * C, H * W)
    out = _sc_unpool(m_t, u_t)
    return out.reshape(B, C, Ho * Wo).transpose(0, 2, 1).reshape(B, Ho, Wo, C)
